# Initial kernel scaffold; baseline (speedup 1.0000x reference)
import jax
import jax.numpy as jnp
from jax.experimental import pallas as pl


def kernel(x, Z, offsets):
    raise NotImplementedError("write your pallas kernel here")


# speedup vs baseline: 573.6448x; 573.6448x over previous
import functools

import jax
import jax.numpy as jnp
from jax import lax
from jax.experimental import pallas as pl
from jax.experimental.pallas import tpu as pltpu
from jax.experimental.pallas import tpu_sc as plsc

_N = 8388608
_N_SPECIES = 100
_TAB = 128
_NC = 2
_NS = 16
_NW = _NC * _NS
_PER_W = _N // _NW
_CHUNK = 16384
_NCHUNK = _PER_W // _CHUNK
_L = 16

_mesh = plsc.VectorSubcoreMesh(core_axis_name="c", subcore_axis_name="s")


@functools.partial(
    pl.kernel,
    out_type=jax.ShapeDtypeStruct((_N,), jnp.float32),
    mesh=_mesh,
    scratch_types=[
        pltpu.VMEM((_TAB,), jnp.float32),
        pltpu.VMEM((2, _CHUNK), jnp.float32),
        pltpu.VMEM((2, _CHUNK), jnp.int32),
        pltpu.VMEM((2, _CHUNK), jnp.float32),
        pltpu.SemaphoreType.DMA,
        pltpu.SemaphoreType.DMA,
        pltpu.SemaphoreType.DMA,
        pltpu.SemaphoreType.DMA,
    ],
    compiler_params=pltpu.CompilerParams(needs_layout_passes=False),
)
def _offset_kernel(x_hbm, z_hbm, off_hbm, out_hbm, tab, xv, zv, yv,
                   si0, si1, so0, so1):
    wid = lax.axis_index("s") * _NC + lax.axis_index("c")
    base = wid * _PER_W

    pltpu.sync_copy(off_hbm, tab)

    in_sems = (si0, si1)
    out_sems = (so0, so1)
    in_descs = [None, None]
    out_descs = [None, None]

    def start_in(g):
        slot = g & 1
        off = base + g * _CHUNK
        d1 = pltpu.async_copy(x_hbm.at[pl.ds(off, _CHUNK)], xv.at[slot],
                              in_sems[slot])
        d2 = pltpu.async_copy(z_hbm.at[pl.ds(off, _CHUNK)], zv.at[slot],
                              in_sems[slot])
        in_descs[slot] = (d1, d2)

    start_in(0)
    for g in range(_NCHUNK):
        slot = g & 1
        if g + 1 < _NCHUNK:
            start_in(g + 1)
        for d in in_descs[slot]:
            d.wait()
        if out_descs[slot] is not None:
            out_descs[slot].wait()

        def body(i, _, slot=slot):
            sl = pl.ds(i * _L, _L)
            idx = zv[slot, sl]
            vals = plsc.load_gather(tab, [idx])
            yv[slot, sl] = xv[slot, sl] + vals
            return 0

        lax.fori_loop(0, _CHUNK // _L, body, 0)

        out_descs[slot] = pltpu.async_copy(
            yv.at[slot], out_hbm.at[pl.ds(base + g * _CHUNK, _CHUNK)],
            out_sems[slot])

    out_descs[0].wait()
    out_descs[1].wait()


def kernel(x, Z, offsets):
    tab = jnp.zeros((_TAB,), jnp.float32).at[:_N_SPECIES].set(offsets)
    return _offset_kernel(x, Z.astype(jnp.int32), tab)


# speedup vs baseline: 974.2523x; 1.6984x over previous
import functools

import jax
import jax.numpy as jnp
from jax import lax
from jax.experimental import pallas as pl
from jax.experimental.pallas import tpu as pltpu
from jax.experimental.pallas import tpu_sc as plsc

_N = 8388608
_N_SPECIES = 100
_TAB = 128
_NC = 2
_NS = 16
_NW = _NC * _NS
_PER_W = _N // _NW
_CHUNK = 16384
_NCHUNK = _PER_W // _CHUNK
_L = 16

_mesh = plsc.VectorSubcoreMesh(core_axis_name="c", subcore_axis_name="s")


@functools.partial(
    pl.kernel,
    out_type=jax.ShapeDtypeStruct((_N,), jnp.float32),
    mesh=_mesh,
    scratch_types=[
        pltpu.VMEM((_TAB,), jnp.float32),
        pltpu.VMEM((2, _CHUNK), jnp.float32),
        pltpu.VMEM((2, _CHUNK), jnp.int32),
        pltpu.VMEM((2, _CHUNK), jnp.float32),
        pltpu.SemaphoreType.DMA,
        pltpu.SemaphoreType.DMA,
        pltpu.SemaphoreType.DMA,
        pltpu.SemaphoreType.DMA,
    ],
    compiler_params=pltpu.CompilerParams(needs_layout_passes=False),
)
def _offset_kernel(x_hbm, z_hbm, off_hbm, out_hbm, tab, xv, zv, yv,
                   si0, si1, so0, so1):
    wid = lax.axis_index("s") * _NC + lax.axis_index("c")
    base = wid * _PER_W

    pltpu.sync_copy(off_hbm, tab)

    in_sems = (si0, si1)
    out_sems = (so0, so1)
    in_descs = [None, None]
    out_descs = [None, None]

    def start_in(g):
        slot = g & 1
        off = base + g * _CHUNK
        d1 = pltpu.async_copy(x_hbm.at[pl.ds(off, _CHUNK)], xv.at[slot],
                              in_sems[slot])
        d2 = pltpu.async_copy(z_hbm.at[pl.ds(off, _CHUNK)], zv.at[slot],
                              in_sems[slot])
        in_descs[slot] = (d1, d2)

    start_in(0)
    for g in range(_NCHUNK):
        slot = g & 1
        if g + 1 < _NCHUNK:
            start_in(g + 1)
        for d in in_descs[slot]:
            d.wait()
        if out_descs[slot] is not None:
            out_descs[slot].wait()

        @plsc.parallel_loop(0, _CHUNK, step=_L, unroll=8)
        def body(i, slot=slot):
            sl = pl.ds(i, _L)
            idx = zv[slot, sl]
            vals = plsc.load_gather(tab, [idx])
            yv[slot, sl] = xv[slot, sl] + vals

        out_descs[slot] = pltpu.async_copy(
            yv.at[slot], out_hbm.at[pl.ds(base + g * _CHUNK, _CHUNK)],
            out_sems[slot])

    out_descs[0].wait()
    out_descs[1].wait()


def kernel(x, Z, offsets):
    tab = jnp.zeros((_TAB,), jnp.float32).at[:_N_SPECIES].set(offsets)
    return _offset_kernel(x, Z.astype(jnp.int32), tab)


# speedup vs baseline: 1406.4381x; 1.4436x over previous
import functools

import jax
import jax.numpy as jnp
from jax import lax
from jax.experimental import pallas as pl
from jax.experimental.pallas import tpu as pltpu
from jax.experimental.pallas import tpu_sc as plsc

_N = 8388608
_N_SPECIES = 100
_TAB = 128
_NC = 2
_NS = 16
_NW = _NC * _NS
_PER_W = _N // _NW
_CHUNK = 16384
_NCHUNK = _PER_W // _CHUNK
_L = 16
_XB = 3
_ZB = 2

_mesh = plsc.VectorSubcoreMesh(core_axis_name="c", subcore_axis_name="s")


@functools.partial(
    pl.kernel,
    out_type=jax.ShapeDtypeStruct((_N,), jnp.float32),
    mesh=_mesh,
    scratch_types=[
        pltpu.VMEM((_TAB,), jnp.float32),
        pltpu.VMEM((_CHUNK,), jnp.float32),
        pltpu.VMEM((_CHUNK,), jnp.float32),
        pltpu.VMEM((_CHUNK,), jnp.float32),
        pltpu.VMEM((_CHUNK,), jnp.int32),
        pltpu.VMEM((_CHUNK,), jnp.int32),
        pltpu.SemaphoreType.DMA,
        pltpu.SemaphoreType.DMA,
        pltpu.SemaphoreType.DMA,
        pltpu.SemaphoreType.DMA,
        pltpu.SemaphoreType.DMA,
        pltpu.SemaphoreType.DMA,
        pltpu.SemaphoreType.DMA,
        pltpu.SemaphoreType.DMA,
    ],
    compiler_params=pltpu.CompilerParams(needs_layout_passes=False),
)
def _offset_kernel(x_hbm, z_hbm, off_hbm, out_hbm, tab, xv0, xv1, xv2,
                   zv0, zv1, xi0, xi1, xi2, zi0, zi1, xo0, xo1, xo2):
    wid = lax.axis_index("s") * _NC + lax.axis_index("c")
    base = wid * _PER_W

    pltpu.sync_copy(off_hbm, tab)

    xvs = (xv0, xv1, xv2)
    zvs = (zv0, zv1)
    x_in_sems = (xi0, xi1, xi2)
    z_in_sems = (zi0, zi1)
    out_sems = (xo0, xo1, xo2)
    x_in_descs = [None] * _XB
    z_in_descs = [None] * _ZB
    out_descs = [None] * _XB

    def start_in(g):
        sx = g % _XB
        sz = g % _ZB
        off = base + g * _CHUNK
        x_in_descs[sx] = pltpu.async_copy(
            x_hbm.at[pl.ds(off, _CHUNK)], xvs[sx], x_in_sems[sx])
        z_in_descs[sz] = pltpu.async_copy(
            z_hbm.at[pl.ds(off, _CHUNK)], zvs[sz], z_in_sems[sz])

    start_in(0)
    for g in range(_NCHUNK):
        sx = g % _XB
        sz = g % _ZB
        if g + 1 < _NCHUNK:
            nx = (g + 1) % _XB
            if out_descs[nx] is not None:
                out_descs[nx].wait()
            start_in(g + 1)
        x_in_descs[sx].wait()
        z_in_descs[sz].wait()

        @plsc.parallel_loop(0, _CHUNK, step=_L, unroll=8)
        def body(i, sx=sx, sz=sz):
            sl = pl.ds(i, _L)
            idx = zvs[sz][sl]
            vals = plsc.load_gather(tab, [idx])
            plsc.addupdate(xvs[sx].at[sl], vals)

        out_descs[sx] = pltpu.async_copy(
            xvs[sx], out_hbm.at[pl.ds(base + g * _CHUNK, _CHUNK)],
            out_sems[sx])

    for d in out_descs:
        if d is not None:
            d.wait()


def kernel(x, Z, offsets):
    tab = jnp.zeros((_TAB,), jnp.float32).at[:_N_SPECIES].set(offsets)
    return _offset_kernel(x, Z.astype(jnp.int32), tab)


# speedup vs baseline: 1464.9420x; 1.0416x over previous
import functools

import jax
import jax.numpy as jnp
from jax import lax
from jax.experimental import pallas as pl
from jax.experimental.pallas import tpu as pltpu
from jax.experimental.pallas import tpu_sc as plsc

_N = 8388608
_N_SPECIES = 100
_TAB = 128
_NC = 2
_NS = 16
_NW = _NC * _NS
_PER_W = _N // _NW
_CHUNK = 16384
_NCHUNK = _PER_W // _CHUNK
_L = 16
_XB = 3
_ZB = 2

_mesh = plsc.VectorSubcoreMesh(core_axis_name="c", subcore_axis_name="s")


@functools.partial(
    pl.kernel,
    out_type=jax.ShapeDtypeStruct((_N,), jnp.float32),
    mesh=_mesh,
    scratch_types=[
        pltpu.VMEM((_N_SPECIES,), jnp.float32),
        pltpu.VMEM((_CHUNK,), jnp.float32),
        pltpu.VMEM((_CHUNK,), jnp.float32),
        pltpu.VMEM((_CHUNK,), jnp.float32),
        pltpu.VMEM((_CHUNK,), jnp.int32),
        pltpu.VMEM((_CHUNK,), jnp.int32),
        pltpu.SemaphoreType.DMA,
        pltpu.SemaphoreType.DMA,
        pltpu.SemaphoreType.DMA,
        pltpu.SemaphoreType.DMA,
        pltpu.SemaphoreType.DMA,
        pltpu.SemaphoreType.DMA,
        pltpu.SemaphoreType.DMA,
        pltpu.SemaphoreType.DMA,
    ],
    compiler_params=pltpu.CompilerParams(needs_layout_passes=False),
)
def _offset_kernel(x_hbm, z_hbm, off_hbm, out_hbm, tab, xv0, xv1, xv2,
                   zv0, zv1, xi0, xi1, xi2, zi0, zi1, xo0, xo1, xo2):
    wid = lax.axis_index("s") * _NC + lax.axis_index("c")
    base = wid * _PER_W

    tab_desc = pltpu.async_copy(off_hbm, tab, xo0)

    xvs = (xv0, xv1, xv2)
    zvs = (zv0, zv1)
    x_in_sems = (xi0, xi1, xi2)
    z_in_sems = (zi0, zi1)
    out_sems = (xo0, xo1, xo2)
    x_in_descs = [None] * _XB
    z_in_descs = [None] * _ZB
    out_descs = [None] * _XB

    def start_in(g):
        sx = g % _XB
        sz = g % _ZB
        off = base + g * _CHUNK
        x_in_descs[sx] = pltpu.async_copy(
            x_hbm.at[pl.ds(off, _CHUNK)], xvs[sx], x_in_sems[sx])
        z_in_descs[sz] = pltpu.async_copy(
            z_hbm.at[pl.ds(off, _CHUNK)], zvs[sz], z_in_sems[sz])

    start_in(0)
    tab_desc.wait()
    for g in range(_NCHUNK):
        sx = g % _XB
        sz = g % _ZB
        if g + 1 < _NCHUNK:
            nx = (g + 1) % _XB
            if out_descs[nx] is not None:
                out_descs[nx].wait()
            start_in(g + 1)
        x_in_descs[sx].wait()
        z_in_descs[sz].wait()

        @plsc.parallel_loop(0, _CHUNK, step=_L, unroll=8)
        def body(i, sx=sx, sz=sz):
            sl = pl.ds(i, _L)
            idx = zvs[sz][sl]
            vals = plsc.load_gather(tab, [idx])
            plsc.addupdate(xvs[sx].at[sl], vals)

        out_descs[sx] = pltpu.async_copy(
            xvs[sx], out_hbm.at[pl.ds(base + g * _CHUNK, _CHUNK)],
            out_sems[sx])

    for d in out_descs:
        if d is not None:
            d.wait()


def kernel(x, Z, offsets):
    return _offset_kernel(x, Z.astype(jnp.int32), offsets)
